# Initial kernel scaffold; baseline (speedup 1.0000x reference)
import jax
import jax.numpy as jnp
from jax.experimental import pallas as pl


def kernel(nf, ef, W_edge, W_node, edge_index):
    raise NotImplementedError("write your pallas kernel here")


# speedup vs baseline: 3.6580x; 3.6580x over previous
import functools

import jax
import jax.numpy as jnp
from jax import lax
from jax.experimental import pallas as pl
from jax.experimental.pallas import tpu as pltpu
from jax.experimental.pallas import tpu_sc as plsc

D = 128
N_PAD = 10240
E_PAD = 327680
NC = 2
NS = 16
BLK = 64
EDGES_PER_TILE = E_PAD // (NC * NS)
NBLK = EDGES_PER_TILE // BLK
ROWS_PER_TILE = N_PAD // NS


def _split_mm_kernel(x_ref, w_ref, o_ref):
    o_ref[...] = jnp.dot(x_ref[...], w_ref[0],
                         preferred_element_type=jnp.float32)


def _tc_split_matmul(x, w2, block_rows):
    rows, k = x.shape
    nh = w2.shape[0]
    dout = w2.shape[2]
    nb = rows // block_rows
    return pl.pallas_call(
        _split_mm_kernel,
        grid=(nh, nb),
        in_specs=[
            pl.BlockSpec((block_rows, k), lambda h, i: (i, 0)),
            pl.BlockSpec((1, k, dout), lambda h, i: (h, 0, 0)),
        ],
        out_specs=pl.BlockSpec((block_rows, dout), lambda h, i: (h * nb + i, 0)),
        out_shape=jax.ShapeDtypeStruct((nh * rows, dout), jnp.float32),
    )(x, w2)


def _mm_kernel(x_ref, w_ref, o_ref):
    o_ref[...] = jnp.dot(x_ref[...], w_ref[...], preferred_element_type=jnp.float32)


def _tc_matmul(x, w, block_rows):
    rows, k = x.shape
    dout = w.shape[1]
    return pl.pallas_call(
        _mm_kernel,
        grid=(rows // block_rows,),
        in_specs=[
            pl.BlockSpec((block_rows, k), lambda i: (i, 0)),
            pl.BlockSpec((k, dout), lambda i: (0, 0)),
        ],
        out_specs=pl.BlockSpec((block_rows, dout), lambda i: (i, 0)),
        out_shape=jax.ShapeDtypeStruct((rows, dout), jnp.float32),
    )(x, w)


def _node_kernel(nf_ref, r0_ref, r1_ref, w1_ref, w2_ref, o_ref):
    r = r0_ref[...] + r1_ref[...]
    y = (jnp.dot(nf_ref[...], w1_ref[...], preferred_element_type=jnp.float32)
         + jnp.dot(r, w2_ref[...], preferred_element_type=jnp.float32))
    o_ref[...] = jnp.maximum(y, jnp.float32(0.01) * y)


def _tc_node_mlp(nf_pad, r0, r1, w1, w2, block_rows=2048):
    rows = nf_pad.shape[0]
    spec = pl.BlockSpec((block_rows, D), lambda i: (i, 0))
    wspec = pl.BlockSpec((D, D), lambda i: (0, 0))
    return pl.pallas_call(
        _node_kernel,
        grid=(rows // block_rows,),
        in_specs=[spec, spec, spec, wspec, wspec],
        out_specs=spec,
        out_shape=jax.ShapeDtypeStruct((rows, D), jnp.float32),
    )(nf_pad, r0, r1, w1, w2)


def _sc_edge_body(tab_hbm, c_hbm, isrc_hbm, idst_hbm, isrcb_hbm, idstb_hbm,
                  out_hbm,
                  sidx, didx, sbidx, dbidx, asb, bdb, adb, bsb, cbuf,
                  accum, sem1, sem2, sem3, sem4, sem5):
    cid = lax.axis_index("c")
    sid = lax.axis_index("s")

    def _zrow(i, carry):
        for j in range(D // 16):
            asb[i, pl.ds(j * 16, 16)] = jnp.zeros((16,), jnp.float32)
        return carry
    lax.fori_loop(0, BLK, _zrow, 0)

    rows0 = sid * ROWS_PER_TILE

    def _zacc(k, carry):
        pltpu.sync_copy(asb, accum.at[pl.ds(rows0 + k * BLK, BLK)])
        return carry
    lax.fori_loop(0, ROWS_PER_TILE // BLK, _zacc, 0)
    plsc.subcore_barrier()

    tile_base = (cid * NS + sid) * EDGES_PER_TILE

    def _block(b, carry):
        base = tile_base + b * BLK
        pltpu.sync_copy(isrc_hbm.at[pl.ds(base, BLK)], sidx)
        pltpu.sync_copy(idst_hbm.at[pl.ds(base, BLK)], didx)
        pltpu.sync_copy(isrcb_hbm.at[pl.ds(base, BLK)], sbidx)
        pltpu.sync_copy(idstb_hbm.at[pl.ds(base, BLK)], dbidx)
        cp1 = pltpu.async_copy(tab_hbm.at[sidx], asb, sem1)
        cp2 = pltpu.async_copy(tab_hbm.at[dbidx], bdb, sem2)
        cp3 = pltpu.async_copy(tab_hbm.at[didx], adb, sem3)
        cp4 = pltpu.async_copy(tab_hbm.at[sbidx], bsb, sem4)
        cp5 = pltpu.async_copy(c_hbm.at[pl.ds(base, BLK)], cbuf, sem5)
        cp1.wait()
        cp2.wait()
        cp3.wait()
        cp4.wait()
        cp5.wait()

        def _edge(e, ecarry):
            for j in range(D // 16):
                lo = j * 16
                a_s = asb[e, pl.ds(lo, 16)]
                b_d = bdb[e, pl.ds(lo, 16)]
                a_d = adb[e, pl.ds(lo, 16)]
                b_s = bsb[e, pl.ds(lo, 16)]
                c = cbuf[e, pl.ds(lo, 16)]
                f = a_s + b_d + c
                r = b_s + a_d + c
                asb[e, pl.ds(lo, 16)] = jnp.maximum(f, jnp.float32(0.01) * f)
                bsb[e, pl.ds(lo, 16)] = jnp.maximum(r, jnp.float32(0.01) * r)
            return ecarry
        lax.fori_loop(0, BLK, _edge, 0)

        pltpu.sync_copy(asb, accum.at[didx], add=True)
        pltpu.sync_copy(bsb, accum.at[sidx], add=True)
        return carry
    lax.fori_loop(0, NBLK, _block, 0)

    plsc.subcore_barrier()
    out_base = cid * N_PAD + rows0

    def _wout(k, carry):
        pltpu.sync_copy(accum.at[pl.ds(rows0 + k * BLK, BLK)], asb)
        pltpu.sync_copy(asb, out_hbm.at[pl.ds(out_base + k * BLK, BLK)])
        return carry
    lax.fori_loop(0, ROWS_PER_TILE // BLK, _wout, 0)


def _sc_edge_pass(tab, c_all, isrc, idst, isrcb, idstb):
    mesh = plsc.VectorSubcoreMesh(core_axis_name="c", subcore_axis_name="s")
    fn = functools.partial(
        pl.kernel,
        mesh=mesh,
        out_type=jax.ShapeDtypeStruct((NC * N_PAD, D), jnp.float32),
        scratch_types=[
            pltpu.VMEM((BLK,), jnp.int32),
            pltpu.VMEM((BLK,), jnp.int32),
            pltpu.VMEM((BLK,), jnp.int32),
            pltpu.VMEM((BLK,), jnp.int32),
            pltpu.VMEM((BLK, D), jnp.float32),
            pltpu.VMEM((BLK, D), jnp.float32),
            pltpu.VMEM((BLK, D), jnp.float32),
            pltpu.VMEM((BLK, D), jnp.float32),
            pltpu.VMEM((BLK, D), jnp.float32),
            pltpu.VMEM_SHARED((N_PAD, D), jnp.float32),
            pltpu.SemaphoreType.DMA,
            pltpu.SemaphoreType.DMA,
            pltpu.SemaphoreType.DMA,
            pltpu.SemaphoreType.DMA,
            pltpu.SemaphoreType.DMA,
        ],
    )(_sc_edge_body)
    return fn(tab, c_all, isrc, idst, isrcb, idstb)


def kernel(nf, ef, W_edge, W_node, edge_index):
    n, d = nf.shape
    e = ef.shape[0]

    src = edge_index[0].astype(jnp.int32)
    dst = edge_index[1].astype(jnp.int32)
    pad_idx = jnp.full((E_PAD - e,), n, dtype=jnp.int32)
    isrc = jnp.concatenate([src, pad_idx])
    idst = jnp.concatenate([dst, pad_idx])
    isrcb = isrc + N_PAD
    idstb = idst + N_PAD

    nf_pad = jnp.pad(nf, ((0, N_PAD - n), (0, 0)))
    ef_pad = jnp.pad(ef, ((0, E_PAD - e), (0, 0)))

    ws_t = W_edge[:, :d].T
    wd_t = W_edge[:, d:2 * d].T
    wc_t = W_edge[:, 2 * d:].T
    w_ab = jnp.stack([ws_t, wd_t])

    tab = _tc_split_matmul(nf_pad, w_ab, block_rows=2048)
    c_all = _tc_matmul(ef_pad, wc_t, block_rows=4096)

    red = _sc_edge_pass(tab, c_all, isrc, idst, isrcb, idstb)
    r0 = red[:N_PAD]
    r1 = red[N_PAD:]

    w_n1 = W_node[:, :d].T
    w_n2 = W_node[:, d:].T

    out = _tc_node_mlp(nf_pad, r0, r1, w_n1, w_n2)
    return out[:n]


# speedup vs baseline: 5.0411x; 1.3781x over previous
import functools

import jax
import jax.numpy as jnp
from jax import lax
from jax.experimental import pallas as pl
from jax.experimental.pallas import tpu as pltpu
from jax.experimental.pallas import tpu_sc as plsc

D = 128
N_PAD = 10240
E_PAD = 327680
NC = 2
NS = 16
BLK = 32
EDGES_PER_TILE = E_PAD // (NC * NS)
NBLK = EDGES_PER_TILE // BLK
ROWS_PER_TILE = N_PAD // NS


def _split_mm_kernel(x_ref, w_ref, o_ref):
    o_ref[...] = jnp.dot(x_ref[...], w_ref[0],
                         preferred_element_type=jnp.float32)


def _tc_split_matmul(x, w2, block_rows):
    rows, k = x.shape
    nh = w2.shape[0]
    dout = w2.shape[2]
    nb = rows // block_rows
    return pl.pallas_call(
        _split_mm_kernel,
        grid=(nh, nb),
        in_specs=[
            pl.BlockSpec((block_rows, k), lambda h, i: (i, 0)),
            pl.BlockSpec((1, k, dout), lambda h, i: (h, 0, 0)),
        ],
        out_specs=pl.BlockSpec((block_rows, dout), lambda h, i: (h * nb + i, 0)),
        out_shape=jax.ShapeDtypeStruct((nh * rows, dout), jnp.float32),
    )(x, w2)


def _mm_kernel(x_ref, w_ref, o_ref):
    o_ref[...] = jnp.dot(x_ref[...], w_ref[...], preferred_element_type=jnp.float32)


def _tc_matmul(x, w, block_rows):
    rows, k = x.shape
    dout = w.shape[1]
    return pl.pallas_call(
        _mm_kernel,
        grid=(rows // block_rows,),
        in_specs=[
            pl.BlockSpec((block_rows, k), lambda i: (i, 0)),
            pl.BlockSpec((k, dout), lambda i: (0, 0)),
        ],
        out_specs=pl.BlockSpec((block_rows, dout), lambda i: (i, 0)),
        out_shape=jax.ShapeDtypeStruct((rows, dout), jnp.float32),
    )(x, w)


def _node_kernel(nf_ref, r0_ref, r1_ref, w1_ref, w2_ref, o_ref):
    r = r0_ref[...] + r1_ref[...]
    y = (jnp.dot(nf_ref[...], w1_ref[...], preferred_element_type=jnp.float32)
         + jnp.dot(r, w2_ref[...], preferred_element_type=jnp.float32))
    o_ref[...] = jnp.maximum(y, jnp.float32(0.01) * y)


def _tc_node_mlp(nf_pad, r0, r1, w1, w2, block_rows=2048):
    rows = nf_pad.shape[0]
    spec = pl.BlockSpec((block_rows, D), lambda i: (i, 0))
    wspec = pl.BlockSpec((D, D), lambda i: (0, 0))
    return pl.pallas_call(
        _node_kernel,
        grid=(rows // block_rows,),
        in_specs=[spec, spec, spec, wspec, wspec],
        out_specs=spec,
        out_shape=jax.ShapeDtypeStruct((rows, D), jnp.float32),
    )(nf_pad, r0, r1, w1, w2)


def _sc_edge_body(tab_hbm, c_hbm, isrc_hbm, idst_hbm, isrcb_hbm, idstb_hbm,
                  out_hbm, *refs):
    (gix0, six0, dix0, asb0, bdb0, adb0, bsb0, cb0,
     gix1, six1, dix1, asb1, bdb1, adb1, bsb1, cb1,
     accum,
     sgx0, ssx0, sg0, ssc0, sgx1, ssx1, sg1, ssc1) = refs
    sets = (
        (gix0, six0, dix0, asb0, bdb0, adb0, bsb0, cb0, sgx0, ssx0, sg0, ssc0),
        (gix1, six1, dix1, asb1, bdb1, adb1, bsb1, cb1, sgx1, ssx1, sg1, ssc1),
    )
    cid = lax.axis_index("c")
    sid = lax.axis_index("s")

    def _zrow(i, carry):
        for j in range(D // 16):
            asb0[i, pl.ds(j * 16, 16)] = jnp.zeros((16,), jnp.float32)
        return carry
    lax.fori_loop(0, BLK, _zrow, 0)

    rows0 = sid * ROWS_PER_TILE
    ZROWS = 128

    def _zacc(k, carry):
        for i in range(ZROWS // BLK):
            pltpu.sync_copy(asb0, accum.at[pl.ds(rows0 + k * ZROWS + i * BLK,
                                                 BLK)])
        return carry
    lax.fori_loop(0, ROWS_PER_TILE // ZROWS, _zacc, 0)
    plsc.subcore_barrier()

    tile_base = (cid * NS + sid) * EDGES_PER_TILE

    def _gidx(b, s, wait):
        gix, semgx = s[0], s[8]
        base = tile_base + b * BLK
        pairs = (
            (isrc_hbm.at[pl.ds(base, BLK)], gix.at[0]),
            (idst_hbm.at[pl.ds(base, BLK)], gix.at[1]),
            (isrcb_hbm.at[pl.ds(base, BLK)], gix.at[2]),
            (idstb_hbm.at[pl.ds(base, BLK)], gix.at[3]),
        )
        for src, dst in pairs:
            if wait:
                pltpu.make_async_copy(src, dst, semgx).wait()
            else:
                pltpu.async_copy(src, dst, semgx)

    def _sidx(b, s, wait):
        six, dix, semsx = s[1], s[2], s[9]
        base = tile_base + b * BLK
        pairs = (
            (isrc_hbm.at[pl.ds(base, BLK)], six),
            (idst_hbm.at[pl.ds(base, BLK)], dix),
        )
        for src, dst in pairs:
            if wait:
                pltpu.make_async_copy(src, dst, semsx).wait()
            else:
                pltpu.async_copy(src, dst, semsx)

    def _gathers(b, s, wait):
        gix, asb, bdb, adb, bsb, cb, semg = s[0], s[3], s[4], s[5], s[6], s[7], s[10]
        base = tile_base + b * BLK
        pairs = (
            (tab_hbm.at[gix.at[0]], asb),
            (tab_hbm.at[gix.at[3]], bdb),
            (tab_hbm.at[gix.at[1]], adb),
            (tab_hbm.at[gix.at[2]], bsb),
            (c_hbm.at[pl.ds(base, BLK)], cb),
        )
        for src, dst in pairs:
            if wait:
                pltpu.make_async_copy(src, dst, semg).wait()
            else:
                pltpu.async_copy(src, dst, semg)

    def _scatter(s, wait):
        six, dix, asb, bsb, semsc = s[1], s[2], s[3], s[6], s[11]
        if wait:
            pltpu.make_async_copy(asb, accum.at[dix], semsc).wait()
            pltpu.make_async_copy(bsb, accum.at[six], semsc).wait()
        else:
            pltpu.async_copy(asb, accum.at[dix], semsc, add=True)
            pltpu.async_copy(bsb, accum.at[six], semsc, add=True)

    def _compute(s):
        asb, bdb, adb, bsb, cb = s[3], s[4], s[5], s[6], s[7]

        def _edge(e, ecarry):
            for j in range(D // 16):
                lo = j * 16
                a_s = asb[e, pl.ds(lo, 16)]
                b_d = bdb[e, pl.ds(lo, 16)]
                a_d = adb[e, pl.ds(lo, 16)]
                b_s = bsb[e, pl.ds(lo, 16)]
                c = cb[e, pl.ds(lo, 16)]
                f = a_s + b_d + c
                r = b_s + a_d + c
                asb[e, pl.ds(lo, 16)] = jnp.maximum(f, jnp.float32(0.01) * f)
                bsb[e, pl.ds(lo, 16)] = jnp.maximum(r, jnp.float32(0.01) * r)
            return ecarry
        lax.fori_loop(0, BLK, _edge, 0)

    def _phase(b, p, first, last):
        s, o = sets[p], sets[1 - p]
        if not last:
            _gidx(b + 1, o, wait=True)
            if not first:
                _scatter(o, wait=True)
            _gathers(b + 1, o, wait=False)
            if not first:
                _sidx(b + 1, o, wait=False)
        else:
            _scatter(o, wait=True)
        _gathers(b, s, wait=True)
        if not last:
            @pl.when(b + 2 <= NBLK - 1)
            def _():
                _gidx(b + 2, s, wait=False)
        _compute(s)
        _sidx(b, s, wait=True)
        _scatter(s, wait=False)

    _gidx(0, sets[0], wait=False)
    _sidx(0, sets[0], wait=False)
    _gidx(1, sets[1], wait=False)
    _sidx(1, sets[1], wait=False)
    _gidx(0, sets[0], wait=True)
    _gathers(0, sets[0], wait=False)

    _phase(0, 0, first=True, last=False)
    _phase(1, 1, first=False, last=False)

    def _body(g, carry):
        b0 = 2 * g
        _phase(b0, 0, first=False, last=False)
        _phase(b0 + 1, 1, first=False, last=False)
        return carry
    lax.fori_loop(1, NBLK // 2 - 1, _body, 0)

    _phase(NBLK - 2, 0, first=False, last=False)
    _phase(NBLK - 1, 1, first=False, last=True)
    _scatter(sets[1], wait=True)

    plsc.subcore_barrier()
    out_base = cid * N_PAD + rows0

    def _wout(k, carry):
        pltpu.sync_copy(accum.at[pl.ds(rows0 + k * BLK, BLK)], asb0)
        pltpu.sync_copy(asb0, out_hbm.at[pl.ds(out_base + k * BLK, BLK)])
        return carry
    lax.fori_loop(0, ROWS_PER_TILE // BLK, _wout, 0)


def _sc_edge_pass(tab, c_all, isrc, idst, isrcb, idstb):
    mesh = plsc.VectorSubcoreMesh(core_axis_name="c", subcore_axis_name="s")
    set_types = [
        pltpu.VMEM((4, BLK), jnp.int32),
        pltpu.VMEM((BLK,), jnp.int32),
        pltpu.VMEM((BLK,), jnp.int32),
        pltpu.VMEM((BLK, D), jnp.float32),
        pltpu.VMEM((BLK, D), jnp.float32),
        pltpu.VMEM((BLK, D), jnp.float32),
        pltpu.VMEM((BLK, D), jnp.float32),
        pltpu.VMEM((BLK, D), jnp.float32),
    ]
    sem_types = [pltpu.SemaphoreType.DMA] * 4
    fn = functools.partial(
        pl.kernel,
        mesh=mesh,
        out_type=jax.ShapeDtypeStruct((NC * N_PAD, D), jnp.float32),
        scratch_types=(
            set_types + set_types
            + [pltpu.VMEM_SHARED((N_PAD, D), jnp.float32)]
            + sem_types + sem_types
        ),
    )(_sc_edge_body)
    return fn(tab, c_all, isrc, idst, isrcb, idstb)


def kernel(nf, ef, W_edge, W_node, edge_index):
    n, d = nf.shape
    e = ef.shape[0]

    src = edge_index[0].astype(jnp.int32)
    dst = edge_index[1].astype(jnp.int32)
    pad_idx = jnp.full((E_PAD - e,), n, dtype=jnp.int32)
    isrc = jnp.concatenate([src, pad_idx])
    idst = jnp.concatenate([dst, pad_idx])
    isrcb = isrc + N_PAD
    idstb = idst + N_PAD

    nf_pad = jnp.pad(nf, ((0, N_PAD - n), (0, 0)))
    ef_pad = jnp.pad(ef, ((0, E_PAD - e), (0, 0)))

    ws_t = W_edge[:, :d].T
    wd_t = W_edge[:, d:2 * d].T
    wc_t = W_edge[:, 2 * d:].T
    w_ab = jnp.stack([ws_t, wd_t])

    tab = _tc_split_matmul(nf_pad, w_ab, block_rows=2048)
    c_all = _tc_matmul(ef_pad, wc_t, block_rows=4096)

    red = _sc_edge_pass(tab, c_all, isrc, idst, isrcb, idstb)
    r0 = red[:N_PAD]
    r1 = red[N_PAD:]

    w_n1 = W_node[:, :d].T
    w_n2 = W_node[:, d:].T

    out = _tc_node_mlp(nf_pad, r0, r1, w_n1, w_n2)
    return out[:n]
